# baseline (speedup 1.0000x reference)
import functools

import jax
import jax.numpy as jnp
from jax import lax
from jax.experimental import pallas as pl
from jax.experimental.pallas import tpu as pltpu
from jax.experimental.pallas import tpu_sc as plsc

H = W = D = 128
HP = WP = DP = 130
NVOX = H * W * D
NPAD = HP * WP * DP
BI, BJ = 4, 8
NBI, NBJ = H // BI, W // BJ
NW = 32
BLOCKS_PW = NBI * NBJ // NW
BV = BI * BJ * D
VREGS = BV // 16
YEXT, XEXT = 16, 20
DPP = 132
ROWW = DPP * 2
STRIPW = XEXT * ROWW
BLK_ALLOC = YEXT * STRIPW + STRIPW + ROWW + 8
UNROLL = 4


def _floor_i32(x):
    t = x.astype(jnp.int32)
    tf = t.astype(jnp.float32)
    return jnp.where(tf > x, t - 1, t)


def _warp_body(tab_hbm, dx_hbm, dy_hbm, dz_hbm, o0_hbm, o1_hbm,
               blk, d0, d1, d2, wx, wy, wz, pk, ov0, ov1,
               fbi, fbv, fbd, sst, sin, sou, sfb):
    wid = lax.axis_index("s") * 2 + lax.axis_index("c")
    iota = lax.iota(jnp.int32, 16)
    zeros = jnp.full((16,), 0, jnp.int32)
    lane_corner = lax.shift_right_logical(iota, 2)
    m_c0 = lane_corner == 0
    m_c1 = lane_corner == 1
    m_c2 = lane_corner == 2
    m_z1 = (iota & 2) != 0
    m_ch1 = (iota & 1) != 0
    m_l15 = iota == 15

    def block_geo(t):
        b = wid * BLOCKS_PW + t
        bi = lax.shift_right_logical(b, 4)
        bj = b & 15
        bi4 = bi * BI
        bj8 = bj * BJ
        ylo = jnp.minimum(jnp.maximum(bi4 - 5, 0), HP - YEXT)
        xlo = jnp.minimum(jnp.maximum(bj8 - 5, 0), WP - XEXT)
        return bi4, bj8, ylo, xlo

    def run_n0(bi4, bj8, di):
        return pl.multiple_of(((bi4 + di) * W + bj8) * D, 8)

    def fire_in(t):
        bi4, bj8, _, _ = block_geo(t)
        for di in range(BI):
            n0 = run_n0(bi4, bj8, di)
            v0 = di * (BJ * D)
            pltpu.async_copy(dx_hbm.at[pl.ds(n0, BJ * D)],
                             d0.at[pl.ds(v0, BJ * D)], sin)
            pltpu.async_copy(dy_hbm.at[pl.ds(n0, BJ * D)],
                             d1.at[pl.ds(v0, BJ * D)], sin)
            pltpu.async_copy(dz_hbm.at[pl.ds(n0, BJ * D)],
                             d2.at[pl.ds(v0, BJ * D)], sin)

    def wait_in(t):
        bi4, bj8, _, _ = block_geo(t)
        for di in range(BI):
            n0 = run_n0(bi4, bj8, di)
            v0 = di * (BJ * D)
            pltpu.make_async_copy(dx_hbm.at[pl.ds(n0, BJ * D)],
                                  d0.at[pl.ds(v0, BJ * D)], sin).wait()
            pltpu.make_async_copy(dy_hbm.at[pl.ds(n0, BJ * D)],
                                  d1.at[pl.ds(v0, BJ * D)], sin).wait()
            pltpu.make_async_copy(dz_hbm.at[pl.ds(n0, BJ * D)],
                                  d2.at[pl.ds(v0, BJ * D)], sin).wait()

    def fire_stage(t):
        _, _, ylo, xlo = block_geo(t)
        for yy in range(YEXT):
            src = pl.multiple_of(((ylo + yy) * WP + xlo) * ROWW, 8)
            pltpu.async_copy(tab_hbm.at[pl.ds(src, STRIPW)],
                             blk.at[pl.ds(yy * STRIPW, STRIPW)], sst)

    def wait_stage(t):
        _, _, ylo, xlo = block_geo(t)
        for yy in range(YEXT):
            src = pl.multiple_of(((ylo + yy) * WP + xlo) * ROWW, 8)
            pltpu.make_async_copy(tab_hbm.at[pl.ds(src, STRIPW)],
                                  blk.at[pl.ds(yy * STRIPW, STRIPW)],
                                  sst).wait()

    def fire_out(t):
        bi4, bj8, _, _ = block_geo(t)
        for di in range(BI):
            n0 = run_n0(bi4, bj8, di)
            v0 = di * (BJ * D)
            pltpu.async_copy(ov0.at[pl.ds(v0, BJ * D)],
                             o0_hbm.at[pl.ds(n0, BJ * D)], sou)
            pltpu.async_copy(ov1.at[pl.ds(v0, BJ * D)],
                             o1_hbm.at[pl.ds(n0, BJ * D)], sou)

    def drain_out(t):
        bi4, bj8, _, _ = block_geo(t)
        for di in range(BI):
            n0 = run_n0(bi4, bj8, di)
            v0 = di * (BJ * D)
            pltpu.make_async_copy(ov0.at[pl.ds(v0, BJ * D)],
                                  o0_hbm.at[pl.ds(n0, BJ * D)], sou).wait()
            pltpu.make_async_copy(ov1.at[pl.ds(v0, BJ * D)],
                                  o1_hbm.at[pl.ds(n0, BJ * D)], sou).wait()

    def pass_a(t):
        bi4, bj8, ylo, xlo = block_geo(t)

        def pa_body(i, bad_acc):
            s = pl.ds(i * 16, 16)
            v = i * 16 + iota
            di = lax.shift_right_logical(v, 10)
            dj = lax.shift_right_logical(v, 7) & 7
            kk = v & 127
            x = (d0[s] + (bj8 + dj).astype(jnp.float32)) + 1.0
            y = (d1[s] + (bi4 + di).astype(jnp.float32)) + 1.0
            z = (d2[s] + kk.astype(jnp.float32)) + 1.0
            x0 = _floor_i32(x)
            y0 = _floor_i32(y)
            z0 = _floor_i32(z)
            x0c = jnp.minimum(jnp.maximum(x0, 0), WP - 1)
            x1c = jnp.minimum(jnp.maximum(x0 + 1, 0), WP - 1)
            y0c = jnp.minimum(jnp.maximum(y0, 0), HP - 1)
            y1c = jnp.minimum(jnp.maximum(y0 + 1, 0), HP - 1)
            z0c = jnp.minimum(jnp.maximum(z0, 0), DP - 1)
            z1c = jnp.minimum(jnp.maximum(z0 + 1, 0), DP - 1)
            wx[s] = x1c.astype(jnp.float32) - x
            wy[s] = y1c.astype(jnp.float32) - y
            wz[s] = z1c.astype(jnp.float32) - z
            mx = x1c - x0c
            my = y1c - y0c
            mz = z1c - z0c
            ylov = zeros + ylo
            yhiv = zeros + (ylo + (YEXT - 1))
            xlov = zeros + xlo
            xhiv = zeros + (xlo + (XEXT - 1))
            b1 = jnp.where(y0c < ylov, 1, 0)
            b2 = jnp.where(y0c + my > yhiv, 1, 0)
            b3 = jnp.where(x0c < xlov, 1, 0)
            b4 = jnp.where(x0c + mx > xhiv, 1, 0)
            badi = (b1 | b2) | (b3 | b4)
            pk[s] = (y0c | (x0c << 8) | (z0c << 16)
                     | (mx << 24) | (my << 25) | (mz << 26) | (badi << 27))
            return bad_acc | badi

        return lax.fori_loop(0, VREGS, pa_body, jnp.zeros((16,), jnp.int32))

    def pass_b(t):
        _, _, ylo, xlo = block_geo(t)

        def pb_body(i, c):
            s = pl.ds(i * 16, 16)
            p = pk[s]
            y0 = p & 255
            x0 = lax.shift_right_logical(p, 8) & 255
            z0 = lax.shift_right_logical(p, 16) & 255
            mx = lax.shift_right_logical(p, 24) & 1
            my = lax.shift_right_logical(p, 25) & 1
            mz = lax.shift_right_logical(p, 26) & 1
            y0l = jnp.minimum(jnp.maximum(y0 - ylo, 0), YEXT - 1)
            x0l = jnp.minimum(jnp.maximum(x0 - xlo, 0), XEXT - 1)
            la = (y0l * XEXT + x0l) * ROWW + (z0 << 1)
            dyo = my * STRIPW
            dxo = mx * ROWW
            zo = mz << 1
            bases = (la, la + dyo, la + dxo, la + dyo + dxo)
            dxw = wx[s]
            dyw = wy[s]
            dzw = wz[s]
            exw = 1.0 - dxw
            eyw = 1.0 - dyw
            ezw = 1.0 - dzw
            wxy = (dxw * dyw, dxw * eyw, exw * dyw, exw * eyw)
            o0 = jnp.full((16,), 0.0, jnp.float32)
            o1 = jnp.full((16,), 0.0, jnp.float32)
            for r in range(4):
                ar = bases[r]
                v00 = plsc.load_gather(blk, [ar])
                v01 = plsc.load_gather(blk, [ar + 1])
                v10 = plsc.load_gather(blk, [ar + zo])
                v11 = plsc.load_gather(blk, [ar + zo + 1])
                w0 = dzw * wxy[r]
                w1 = ezw * wxy[r]
                o0 = o0 + (w0 * v00 + w1 * v10)
                o1 = o1 + (w0 * v01 + w1 * v11)
            ov0[s] = o0
            ov1[s] = o1
            return c

        lax.fori_loop(0, VREGS, pb_body, 0)

    def fallback(t):
        bi4, bj8, _, _ = block_geo(t)

        def scan_vreg(i, c):
            pv = pk[pl.ds(i * 16, 16)]
            vb = lax.shift_right_logical(pv, 27) & 1
            anyb = jnp.max(vb)

            @pl.when(anyb == 1)
            def _():
                def scan_lane(l, c2):
                    v = i * 16 + l
                    pvb = lax.shift_right_logical(
                        plsc.load_gather(pk, [jnp.full((16,), v, jnp.int32)]),
                        27) & 1
                    bsc = jnp.max(pvb)

                    @pl.when(bsc == 1)
                    def _():
                        di = lax.shift_right_logical(v, 10)
                        dj = lax.shift_right_logical(v, 7) & 7
                        kk = v & 127
                        nglob = ((bi4 + di) * W + (bj8 + dj)) * D + kk
                        na = pl.multiple_of(nglob & ~7, 8)
                        off = jnp.full((16,), nglob - na, jnp.int32)
                        pltpu.async_copy(dx_hbm.at[pl.ds(na, 16)],
                                         fbd, sfb).wait()
                        dxs = plsc.load_gather(fbd, [off])
                        pltpu.async_copy(dy_hbm.at[pl.ds(na, 16)],
                                         fbd, sfb).wait()
                        dys = plsc.load_gather(fbd, [off])
                        pltpu.async_copy(dz_hbm.at[pl.ds(na, 16)],
                                         fbd, sfb).wait()
                        dzs = plsc.load_gather(fbd, [off])
                        x = (dxs + (bj8 + dj).astype(jnp.float32)) + 1.0
                        y = (dys + (bi4 + di).astype(jnp.float32)) + 1.0
                        z = (dzs + kk.astype(jnp.float32)) + 1.0
                        x0 = _floor_i32(x)
                        y0 = _floor_i32(y)
                        z0 = _floor_i32(z)
                        x0c = jnp.minimum(jnp.maximum(x0, 0), WP - 1)
                        x1c = jnp.minimum(jnp.maximum(x0 + 1, 0), WP - 1)
                        y0c = jnp.minimum(jnp.maximum(y0, 0), HP - 1)
                        y1c = jnp.minimum(jnp.maximum(y0 + 1, 0), HP - 1)
                        z0c = jnp.minimum(jnp.maximum(z0, 0), DP - 1)
                        z1c = jnp.minimum(jnp.maximum(z0 + 1, 0), DP - 1)
                        dxw = x1c.astype(jnp.float32) - x
                        dyw = y1c.astype(jnp.float32) - y
                        dzw = z1c.astype(jnp.float32) - z
                        ba = (y0c * WP + x0c) * ROWW + (z0c << 1)
                        bb = (y1c * WP + x0c) * ROWW + (z0c << 1)
                        bc = (y0c * WP + x1c) * ROWW + (z0c << 1)
                        bd = (y1c * WP + x1c) * ROWW + (z0c << 1)
                        zo = (z1c - z0c) << 1
                        base = jnp.where(m_c0, ba,
                                         jnp.where(m_c1, bb,
                                                   jnp.where(m_c2, bc, bd)))
                        addr = (base + jnp.where(m_z1, zo, 0)
                                + (iota & 1))
                        fbi[pl.ds(0, 16)] = addr
                        pltpu.async_copy(tab_hbm.at[fbi], fbv, sfb).wait()
                        vals = fbv[pl.ds(0, 16)]
                        exw = 1.0 - dxw
                        eyw = 1.0 - dyw
                        ezw = 1.0 - dzw
                        wxy0 = dxw * dyw
                        wxy1 = dxw * eyw
                        wxy2 = exw * dyw
                        wxy3 = exw * eyw
                        wyx = jnp.where(m_c0, wxy0,
                                        jnp.where(m_c1, wxy1,
                                                  jnp.where(m_c2, wxy2,
                                                            wxy3)))
                        wzv = jnp.where(m_z1, ezw, dzw)
                        prod = wyx * wzv * vals
                        o0c = plsc.cumsum(jnp.where(m_ch1, 0.0, prod))
                        o1c = plsc.cumsum(jnp.where(m_ch1, prod, 0.0))
                        idxv = jnp.full((16,), v, jnp.int32)
                        plsc.store_scatter(ov0, [idxv], o0c, mask=m_l15)
                        plsc.store_scatter(ov1, [idxv], o1c, mask=m_l15)
                    return c2

                lax.fori_loop(0, 16, scan_lane, 0)
            return c

        lax.fori_loop(0, VREGS, scan_vreg, 0)

    fire_in(0)

    def block_body(t, carry):
        fire_stage(t)
        wait_in(t)
        bad_flag = pass_a(t)
        fire_in(jnp.minimum(t + 1, BLOCKS_PW - 1))

        @pl.when(t > 0)
        def _():
            drain_out(t - 1)

        wait_stage(t)
        pass_b(t)

        @pl.when(jnp.max(bad_flag) > 0)
        def _():
            fallback(t)

        fire_out(t)
        return carry

    lax.fori_loop(0, BLOCKS_PW, block_body, 0)
    drain_out(BLOCKS_PW - 1)
    wait_in(BLOCKS_PW - 1)


@jax.jit
def _warp(tab, dxf, dyf, dzf):
    mesh = plsc.VectorSubcoreMesh(core_axis_name="c", subcore_axis_name="s")
    f32 = jnp.float32
    i32 = jnp.int32
    scratch = [
        pltpu.VMEM((BLK_ALLOC,), f32),
        pltpu.VMEM((BV,), f32), pltpu.VMEM((BV,), f32),
        pltpu.VMEM((BV,), f32),
        pltpu.VMEM((BV,), f32), pltpu.VMEM((BV,), f32),
        pltpu.VMEM((BV,), f32),
        pltpu.VMEM((BV,), i32),
        pltpu.VMEM((BV,), f32), pltpu.VMEM((BV,), f32),
        pltpu.VMEM((16,), i32), pltpu.VMEM((16,), f32),
        pltpu.VMEM((16,), f32),
        pltpu.SemaphoreType.DMA, pltpu.SemaphoreType.DMA,
        pltpu.SemaphoreType.DMA, pltpu.SemaphoreType.DMA,
    ]
    run = functools.partial(
        pl.kernel,
        mesh=mesh,
        out_type=[jax.ShapeDtypeStruct((NVOX,), f32),
                  jax.ShapeDtypeStruct((NVOX,), f32)],
        scratch_types=scratch,
        compiler_params=pltpu.CompilerParams(needs_layout_passes=False),
    )(_warp_body)
    return run(tab, dxf, dyf, dzf)


def kernel(I, dx_t, dy_t, dz_t):
    I_pad = jnp.pad(I, ((0, 0), (0, 0), (1, 1), (1, 1), (1, 3)))
    tab = jnp.transpose(I_pad[0], (1, 2, 3, 0)).reshape(-1)
    zpad = jnp.zeros((16,), jnp.float32)
    dxf = jnp.concatenate([dx_t.reshape(-1), zpad])
    dyf = jnp.concatenate([dy_t.reshape(-1), zpad])
    dzf = jnp.concatenate([dz_t.reshape(-1), zpad])
    o0, o1 = _warp(tab, dxf, dyf, dzf)
    return jnp.stack([o0, o1]).reshape(1, 2, H, W, D)
